# baseline (speedup 1.0000x reference)
import functools

import jax
import jax.numpy as jnp
from jax import lax
from jax.experimental import pallas as pl
from jax.experimental.pallas import tpu as pltpu
from jax.experimental.pallas import tpu_sc as plsc

EMBED_DIM = 64
SCALE = 8.0
NUM_WORKERS = 32
IDX_ROW = 128
CHUNK_IDX_ROWS = 8
CHUNK = CHUNK_IDX_ROWS * IDX_ROW


def _emb_kernel(n_chunks):
    mesh = plsc.VectorSubcoreMesh(core_axis_name="c", subcore_axis_name="s")
    total_rows = NUM_WORKERS * n_chunks * CHUNK

    @functools.partial(
        pl.kernel,
        mesh=mesh,
        out_type=jax.ShapeDtypeStruct((total_rows, EMBED_DIM), jnp.float32),
        scratch_types=[
            pltpu.VMEM((CHUNK_IDX_ROWS, IDX_ROW), jnp.int32),
            pltpu.VMEM((CHUNK, EMBED_DIM), jnp.float32),
            pltpu.SemaphoreType.DMA,
        ],
        compiler_params=pltpu.CompilerParams(use_tc_tiling_on_sc=False),
    )
    def emb(idx_hbm, table_hbm, out_hbm, idx_v, rows_v, sem):
        wid = lax.axis_index("s") * 2 + lax.axis_index("c")
        base = wid * n_chunks

        def chunk_body(ci, carry):
            chunk_no = base + ci
            pltpu.sync_copy(
                idx_hbm.at[pl.ds(chunk_no * CHUNK_IDX_ROWS, CHUNK_IDX_ROWS)],
                idx_v,
            )
            copies = []
            for j in range(CHUNK_IDX_ROWS):
                copies.append(
                    pltpu.async_copy(
                        table_hbm.at[idx_v.at[j]],
                        rows_v.at[pl.ds(j * IDX_ROW, IDX_ROW)],
                        sem,
                    )
                )
            for c in copies:
                c.wait()

            def scale_body(r, c2):
                for d in range(EMBED_DIM // 16):
                    sl = pl.ds(d * 16, 16)
                    rows_v[r, sl] = rows_v[r, sl] * SCALE
                return c2

            lax.fori_loop(0, CHUNK, scale_body, 0, unroll=4)

            pltpu.sync_copy(rows_v, out_hbm.at[pl.ds(chunk_no * CHUNK, CHUNK)])
            return carry

        lax.fori_loop(0, n_chunks, chunk_body, 0)

    return emb


def kernel(x, table):
    b, s = x.shape
    total = b * s
    assert total % (NUM_WORKERS * CHUNK) == 0
    n_chunks = total // (NUM_WORKERS * CHUNK)
    idx = x.reshape(total // IDX_ROW, IDX_ROW).astype(jnp.int32)
    out = _emb_kernel(n_chunks)(idx, table)
    return out.reshape(b, s, EMBED_DIM)


# speedup vs baseline: 1.0585x; 1.0585x over previous
import functools

import jax
import jax.numpy as jnp
from jax import lax
from jax.experimental import pallas as pl
from jax.experimental.pallas import tpu as pltpu
from jax.experimental.pallas import tpu_sc as plsc

EMBED_DIM = 64
SCALE = 8.0
NUM_WORKERS = 32
IDX_ROW = 128
SPC = 2
CHUNK = SPC * IDX_ROW
NBUF = 4
LOOKAHEAD = 2


def _emb_kernel(n_chunks_per_worker):
    n = n_chunks_per_worker
    assert n % NBUF == 0 and n >= 2 * NBUF
    mesh = plsc.VectorSubcoreMesh(core_axis_name="c", subcore_axis_name="s")
    total_rows = NUM_WORKERS * n * CHUNK

    @functools.partial(
        pl.kernel,
        mesh=mesh,
        out_type=jax.ShapeDtypeStruct((total_rows, EMBED_DIM), jnp.float32),
        scratch_types=[
            pltpu.VMEM((NBUF * SPC, IDX_ROW), jnp.int32),
            pltpu.VMEM((NBUF, CHUNK, EMBED_DIM), jnp.float32),
            pltpu.SemaphoreType.DMA((NBUF,)),
            pltpu.SemaphoreType.DMA((NBUF,)),
        ],
        compiler_params=pltpu.CompilerParams(use_tc_tiling_on_sc=False),
    )
    def emb(idx_hbm, table_hbm, out_hbm, idx_v, rows_v, sem_g, sem_o):
        wid = lax.axis_index("s") * 2 + lax.axis_index("c")
        base = wid * n

        def fire_gathers(chunk, slot):
            pltpu.sync_copy(
                idx_hbm.at[pl.ds((base + chunk) * SPC, SPC)],
                idx_v.at[pl.ds(slot * SPC, SPC)],
            )
            for j in range(SPC):
                pltpu.async_copy(
                    table_hbm.at[idx_v.at[slot * SPC + j]],
                    rows_v.at[slot].at[pl.ds(j * IDX_ROW, IDX_ROW)],
                    sem_g.at[slot],
                )

        def wait_gathers(chunk, slot):
            for j in range(SPC):
                pltpu.make_async_copy(
                    table_hbm.at[idx_v.at[slot * SPC + j]],
                    rows_v.at[slot].at[pl.ds(j * IDX_ROW, IDX_ROW)],
                    sem_g.at[slot],
                ).wait()

        def store_chunk(chunk, slot):
            pltpu.async_copy(
                rows_v.at[slot],
                out_hbm.at[pl.ds((base + chunk) * CHUNK, CHUNK)],
                sem_o.at[slot],
            )

        def wait_store(chunk, slot):
            pltpu.make_async_copy(
                rows_v.at[slot],
                out_hbm.at[pl.ds((base + chunk) * CHUNK, CHUNK)],
                sem_o.at[slot],
            ).wait()

        for k in range(LOOKAHEAD):
            fire_gathers(k, k)

        def group_body(g0, carry):
            for b in range(NBUF):
                g = g0 * NBUF + b
                pf = g + LOOKAHEAD
                slot_pf = (b + LOOKAHEAD) % NBUF

                @pl.when(pf < n)
                def _():
                    @pl.when(pf >= NBUF)
                    def _():
                        wait_store(pf - NBUF, slot_pf)

                    fire_gathers(pf, slot_pf)

                wait_gathers(g, b)

                def scale_body(r, c2):
                    for d in range(EMBED_DIM // 16):
                        sl = pl.ds(d * 16, 16)
                        rows_v[b, r, sl] = rows_v[b, r, sl] * SCALE
                    return c2

                lax.fori_loop(0, CHUNK, scale_body, 0, unroll=8)
                store_chunk(g, b)
            return carry

        lax.fori_loop(0, n // NBUF, group_body, 0)

        for b in range(NBUF):
            wait_store(n - NBUF + b, (n - NBUF + b) % NBUF)

    return emb


def kernel(x, table):
    b, s = x.shape
    total = b * s
    assert total % (NUM_WORKERS * CHUNK) == 0
    n_chunks = total // (NUM_WORKERS * CHUNK)
    idx = x.reshape(total // IDX_ROW, IDX_ROW).astype(jnp.int32)
    out = _emb_kernel(n_chunks)(idx, table)
    return out.reshape(b, s, EMBED_DIM)
